# baseline (speedup 1.0000x reference)
import functools

import jax
import jax.numpy as jnp
import numpy as np
from jax import lax
from jax.experimental import pallas as pl
from jax.experimental.pallas import tpu as pltpu
from jax.experimental.pallas import tpu_sc as plsc

_B, _F, _V, _E = 16384, 26, 100000, 16
_NC, _NS = 2, 16
_NW = _NC * _NS
_G = 128
_BF = _B * _F
_PER_W = _BF // _NW
_NGRP = _PER_W // _G
_BB = 512


@functools.cache
def _make_sc_gather():
    @functools.partial(
        pl.kernel,
        mesh=plsc.VectorSubcoreMesh(core_axis_name="c", subcore_axis_name="s"),
        out_type=(
            jax.ShapeDtypeStruct((_BF, _E), jnp.float32),
            jax.ShapeDtypeStruct((_BF,), jnp.float32),
        ),
        scratch_types=[
            pltpu.VMEM((_NGRP, _G), jnp.int32),
            pltpu.VMEM((_G, _E), jnp.float32),
            pltpu.VMEM((_G,), jnp.float32),
            pltpu.SemaphoreType.DMA,
            pltpu.SemaphoreType.DMA,
        ],
        compiler_params=pltpu.CompilerParams(use_tc_tiling_on_sc=False),
    )
    def _sc_gather(idx_hbm, emb2_hbm, emb1_hbm, e2_out, e1_out,
                   idx_v, rows_v, vals_v, sem2, sem1):
        wid = lax.axis_index("s") * _NC + lax.axis_index("c")
        base = wid * _PER_W
        pltpu.sync_copy(idx_hbm.at[wid], idx_v)

        def body(g, carry):
            cp2 = pltpu.async_copy(emb2_hbm.at[idx_v.at[g]], rows_v, sem2)
            cp1 = pltpu.async_copy(emb1_hbm.at[idx_v.at[g]], vals_v, sem1)
            cp2.wait()
            cp1.wait()
            pltpu.sync_copy(rows_v, e2_out.at[pl.ds(base + g * _G, _G)])
            pltpu.sync_copy(vals_v, e1_out.at[pl.ds(base + g * _G, _G)])
            return carry

        lax.fori_loop(0, _NGRP, body, 0)

    return _sc_gather


def _tc_body(e2_ref, xv_ref, e1_ref, r_ref, m1_ref, w1_ref, b1_ref,
             w2_ref, b2_ref, bias_ref, out_ref):
    e2 = e2_ref[...]
    xv = xv_ref[...]
    xvrep = jnp.dot(xv, r_ref[...],
                    preferred_element_type=jnp.float32)
    deep = e2 * xvrep
    fm_sum = jnp.dot(deep, m1_ref[...],
                     preferred_element_type=jnp.float32)
    fm2 = 0.5 * (jnp.sum(fm_sum * fm_sum, axis=1)
                 - jnp.sum(deep * deep, axis=1))
    fm1 = jnp.sum(e1_ref[...] * xv, axis=1)
    h = jnp.maximum(
        jnp.dot(deep, w1_ref[...], preferred_element_type=jnp.float32)
        + b1_ref[...], 0.0)
    h2 = jnp.maximum(
        jnp.dot(h, w2_ref[...], preferred_element_type=jnp.float32)
        + b2_ref[...], 0.0)
    out_ref[...] = (fm1 + fm2 + jnp.sum(h2, axis=1)
                    + bias_ref[0, 0])[None, None, :]


_R_NP = np.kron(np.eye(_F), np.ones((1, _E))).astype(np.float32)
_M1_NP = np.kron(np.ones((_F, 1)), np.eye(_E)).astype(np.float32)


def _tc_dense(e2m, Xv, e1m, W1, b1, W2, b2, bias):
    nblk = _B // _BB
    out = pl.pallas_call(
        _tc_body,
        grid=(nblk,),
        in_specs=[
            pl.BlockSpec((_BB, _F * _E), lambda i: (i, 0)),
            pl.BlockSpec((_BB, _F), lambda i: (i, 0)),
            pl.BlockSpec((_BB, _F), lambda i: (i, 0)),
            pl.BlockSpec((_F, _F * _E), lambda i: (0, 0)),
            pl.BlockSpec((_F * _E, _E), lambda i: (0, 0)),
            pl.BlockSpec((_F * _E, 128), lambda i: (0, 0)),
            pl.BlockSpec((1, 128), lambda i: (0, 0)),
            pl.BlockSpec((128, 128), lambda i: (0, 0)),
            pl.BlockSpec((1, 128), lambda i: (0, 0)),
            pl.BlockSpec((1, 1), lambda i: (0, 0)),
        ],
        out_specs=pl.BlockSpec((1, 1, _BB), lambda i: (i, 0, 0)),
        out_shape=jax.ShapeDtypeStruct((nblk, 1, _BB), jnp.float32),
    )(e2m, Xv, e1m, jnp.asarray(_R_NP), jnp.asarray(_M1_NP), W1,
      b1.reshape(1, 128), W2,
      b2.reshape(1, 128), bias.reshape(1, 1))
    return out.reshape(_B)


def kernel(Xi, Xv, emb1, emb2, W1, b1, W2, b2, bias):
    idx = Xi[:, :, 0].astype(jnp.int32)
    flat_idx = idx + (jnp.arange(_F, dtype=jnp.int32) * _V)[None, :]
    idx3 = flat_idx.reshape(_NW, _NGRP, _G)
    emb2_flat = emb2.reshape(_F * _V, _E)
    emb1_flat = emb1.reshape(_F * _V)

    e2_rows, e1_vals = _make_sc_gather()(idx3, emb2_flat, emb1_flat)

    e2m = e2_rows.reshape(_B, _F * _E)
    e1m = e1_vals.reshape(_B, _F)
    return _tc_dense(e2m, Xv, e1m, W1, b1, W2, b2, bias)
